# baseline (speedup 1.0000x reference)
import functools

import jax
import jax.numpy as jnp
from jax import lax
from jax.experimental import pallas as pl
from jax.experimental.pallas import tpu as pltpu
from jax.experimental.pallas import tpu_sc as plsc

NC = 2
NS = 16
LANES = 16
CHUNK = 128


def _make_deg_kernel(npad, n_chunks):
    rows = npad // NS
    mesh = plsc.VectorSubcoreMesh(core_axis_name="c", subcore_axis_name="s")

    @functools.partial(
        pl.kernel,
        mesh=mesh,
        out_type=jax.ShapeDtypeStruct((npad,), jnp.float32),
        scratch_types=[
            pltpu.VMEM((n_chunks, CHUNK), jnp.int32),
            pltpu.VMEM((CHUNK,), jnp.float32),
            pltpu.VMEM((rows,), jnp.float32),
            pltpu.VMEM_SHARED((npad,), jnp.float32),
        ],
    )
    def deg_kernel(dst_hbm, deg_hbm, idx_v, ones_v, buf_v, acc_sh):
        cid = lax.axis_index("c")
        sid = lax.axis_index("s")

        @pl.when(cid == 1)
        def _():
            def zero_body(k, _):
                buf_v[pl.ds(k * LANES, LANES)] = jnp.zeros((LANES,), jnp.float32)
                return _

            lax.fori_loop(0, rows // LANES, zero_body, None)
            pltpu.sync_copy(buf_v, acc_sh.at[pl.ds(sid * rows, rows)])

            def ones_body(k, _):
                ones_v[pl.ds(k * LANES, LANES)] = jnp.full((LANES,), 1.0, jnp.float32)
                return _

            lax.fori_loop(0, CHUNK // LANES, ones_body, None)
            pltpu.sync_copy(dst_hbm.at[sid], idx_v)
            plsc.subcore_barrier()

            def edge_body(j, _):
                pltpu.sync_copy(ones_v, acc_sh.at[idx_v.at[j]], add=True)
                return _

            lax.fori_loop(0, n_chunks, edge_body, None)
            plsc.subcore_barrier()
            pltpu.sync_copy(
                acc_sh.at[pl.ds(sid * rows, rows)],
                deg_hbm.at[pl.ds(sid * rows, rows)],
            )

    return deg_kernel


def _make_agg_kernel(npad, feat, n_chunks):
    rows = npad // NS
    assert n_chunks % 4 == 0
    mesh = plsc.VectorSubcoreMesh(core_axis_name="c", subcore_axis_name="s")

    @functools.partial(
        pl.kernel,
        mesh=mesh,
        out_type=jax.ShapeDtypeStruct((npad, feat), jnp.float32),
        scratch_types=[
            pltpu.VMEM((n_chunks, CHUNK), jnp.int32),
            pltpu.VMEM((n_chunks, CHUNK), jnp.int32),
            [pltpu.VMEM((CHUNK, feat), jnp.float32)] * 4,
            [pltpu.SemaphoreType.DMA] * 4,
            [pltpu.SemaphoreType.DMA] * 4,
            pltpu.VMEM_SHARED((npad, feat), jnp.float32),
        ],
        compiler_params=pltpu.CompilerParams(use_tc_tiling_on_sc=False),
    )
    def agg_kernel(g_hbm, src_hbm, dst_hbm, acc_hbm, src_v, dst_v, bufs, gsem, ssem, acc_sh):
        cid = lax.axis_index("c")
        sid = lax.axis_index("s")
        base = sid * rows

        @pl.when(cid == 0)
        def _body():
            pltpu.sync_copy(g_hbm.at[pl.ds(base, rows)], acc_sh.at[pl.ds(base, rows)])
            pltpu.sync_copy(src_hbm.at[sid], src_v)
            pltpu.sync_copy(dst_hbm.at[sid], dst_v)
            plsc.subcore_barrier()

            for b in range(2):
                pltpu.async_copy(g_hbm.at[src_v.at[b]], bufs[b], gsem[b])

            def edge_body(j0, _):
                for b in range(4):
                    j = j0 * 4 + b
                    b2 = (b + 2) % 4
                    pltpu.make_async_copy(g_hbm.at[src_v.at[j]], bufs[b], gsem[b]).wait()
                    pltpu.async_copy(bufs[b], acc_sh.at[dst_v.at[j]], ssem[b], add=True)

                    @pl.when(j >= 2)
                    def _():
                        pltpu.make_async_copy(
                            bufs[b2], acc_sh.at[dst_v.at[j - 2]], ssem[b2]
                        ).wait()

                    @pl.when(j + 2 < n_chunks)
                    def _():
                        pltpu.async_copy(g_hbm.at[src_v.at[j + 2]], bufs[b2], gsem[b2])
                return _

            lax.fori_loop(0, n_chunks // 4, edge_body, None)
            for b in (2, 3):
                j = n_chunks - 4 + b
                pltpu.make_async_copy(bufs[b], acc_sh.at[dst_v.at[j]], ssem[b]).wait()
            plsc.subcore_barrier()
            pltpu.sync_copy(acc_sh.at[pl.ds(base, rows)], acc_hbm.at[pl.ds(base, rows)])

    return agg_kernel


def _tc_linear(x, w, deg):
    def body(x_ref, w_ref, deg_ref, g_ref, d_ref):
        d_ref[...] = lax.rsqrt(deg_ref[...] + 1.0)
        h = jnp.dot(x_ref[...], w_ref[...], preferred_element_type=jnp.float32)
        g_ref[...] = h * d_ref[...]

    npad = x.shape[0]
    return pl.pallas_call(
        body,
        out_shape=(
            jax.ShapeDtypeStruct((npad, w.shape[1]), jnp.float32),
            jax.ShapeDtypeStruct((npad, 1), jnp.float32),
        ),
    )(x, w, deg)


def _tc_mid(acc, g1, dinv, b1, wcat):
    def body(a_ref, g_ref, d_ref, b_ref, w_ref, o_ref):
        h = jnp.maximum(a_ref[...] * d_ref[...] + b_ref[...], 0.0)
        o_ref[...] = (
            jnp.dot(h, w_ref[...], preferred_element_type=jnp.float32) * d_ref[...]
        )

    return pl.pallas_call(
        body,
        out_shape=jax.ShapeDtypeStruct(g1.shape, jnp.float32),
    )(acc, g1, dinv, b1, wcat)


def _tc_final(acc, g2, dinv, bcat):
    def body(a_ref, g_ref, d_ref, b_ref, o_ref):
        o_ref[...] = a_ref[...] * d_ref[...] + b_ref[...]

    return pl.pallas_call(
        body,
        out_shape=jax.ShapeDtypeStruct(g2.shape, jnp.float32),
    )(acc, g2, dinv, bcat)


def kernel(x, edge_index, W1, b1, W_mu, b_mu, W_logstd, b_logstd):
    n = x.shape[0]
    e = edge_index.shape[1]
    npad = ((n + 16 * 32 - 1) // (16 * 32)) * (16 * 32)

    ei = edge_index.astype(jnp.int32)
    src, dst = ei[0], ei[1]

    n_ch = -(-e // (NS * CHUNK))
    n_ch = ((n_ch + 3) // 4) * 4
    epad = NS * CHUNK * n_ch
    pad = jnp.full((epad - e,), n, jnp.int32)
    srcp = jnp.concatenate([src, pad]).reshape(NS, n_ch, CHUNK)
    dstp = jnp.concatenate([dst, pad]).reshape(NS, n_ch, CHUNK)

    n_chd = -(-e // (NS * CHUNK))
    epadd = NS * CHUNK * n_chd
    dstd = jnp.concatenate([dst, jnp.full((epadd - e,), n, jnp.int32)]).reshape(
        NS, n_chd, CHUNK
    )

    xp = jnp.pad(x, ((0, npad - n), (0, 0)))
    wcat = jnp.concatenate([W_mu, W_logstd], axis=1)
    bcat = jnp.concatenate([b_mu, b_logstd]).reshape(1, -1)

    deg = _make_deg_kernel(npad, n_chd)(dstd).reshape(npad, 1)
    agg = _make_agg_kernel(npad, W1.shape[1], n_ch)

    g1, dinv = _tc_linear(xp, W1, deg)
    acc1 = agg(g1, srcp, dstp)
    g2 = _tc_mid(acc1, g1, dinv, b1.reshape(1, -1), wcat)
    acc2 = agg(g2, srcp, dstp)
    outc = _tc_final(acc2, g2, dinv, bcat)

    o = W_mu.shape[1]
    return (outc[:n, :o], outc[:n, o:])


# speedup vs baseline: 1.0011x; 1.0011x over previous
import functools

import jax
import jax.numpy as jnp
from jax import lax
from jax.experimental import pallas as pl
from jax.experimental.pallas import tpu as pltpu
from jax.experimental.pallas import tpu_sc as plsc

NC = 2
NS = 16
LANES = 16
CHUNK = 128


def _make_deg_kernel(npad, n_chunks):
    rows = npad // NS
    mesh = plsc.VectorSubcoreMesh(core_axis_name="c", subcore_axis_name="s")

    @functools.partial(
        pl.kernel,
        mesh=mesh,
        out_type=jax.ShapeDtypeStruct((npad,), jnp.float32),
        scratch_types=[
            pltpu.VMEM((n_chunks, CHUNK), jnp.int32),
            pltpu.VMEM((CHUNK,), jnp.float32),
            pltpu.VMEM((rows,), jnp.float32),
            pltpu.VMEM_SHARED((npad,), jnp.float32),
        ],
    )
    def deg_kernel(dst_hbm, deg_hbm, idx_v, ones_v, buf_v, acc_sh):
        cid = lax.axis_index("c")
        sid = lax.axis_index("s")

        @pl.when(cid == 1)
        def _():
            def zero_body(k, _):
                buf_v[pl.ds(k * LANES, LANES)] = jnp.zeros((LANES,), jnp.float32)
                return _

            lax.fori_loop(0, rows // LANES, zero_body, None)
            pltpu.sync_copy(buf_v, acc_sh.at[pl.ds(sid * rows, rows)])

            def ones_body(k, _):
                ones_v[pl.ds(k * LANES, LANES)] = jnp.full((LANES,), 1.0, jnp.float32)
                return _

            lax.fori_loop(0, CHUNK // LANES, ones_body, None)
            pltpu.sync_copy(dst_hbm.at[sid], idx_v)
            plsc.subcore_barrier()

            def edge_body(j, _):
                pltpu.sync_copy(ones_v, acc_sh.at[idx_v.at[j]], add=True)
                return _

            lax.fori_loop(0, n_chunks, edge_body, None)
            plsc.subcore_barrier()
            pltpu.sync_copy(
                acc_sh.at[pl.ds(sid * rows, rows)],
                deg_hbm.at[pl.ds(sid * rows, rows)],
            )

    return deg_kernel


def _make_agg_kernel(npad, feat, n_chunks):
    rows = npad // NS
    assert n_chunks % 4 == 0
    mesh = plsc.VectorSubcoreMesh(core_axis_name="c", subcore_axis_name="s")

    @functools.partial(
        pl.kernel,
        mesh=mesh,
        out_type=jax.ShapeDtypeStruct((npad, feat), jnp.float32),
        scratch_types=[
            pltpu.VMEM((n_chunks, CHUNK), jnp.int32),
            pltpu.VMEM((n_chunks, CHUNK), jnp.int32),
            [pltpu.VMEM((CHUNK, feat), jnp.float32)] * 4,
            [pltpu.SemaphoreType.DMA] * 4,
            [pltpu.SemaphoreType.DMA] * 4,
            pltpu.VMEM_SHARED((npad, feat), jnp.float32),
        ],
        compiler_params=pltpu.CompilerParams(use_tc_tiling_on_sc=False),
    )
    def agg_kernel(g_hbm, src_hbm, dst_hbm, acc_hbm, src_v, dst_v, bufs, gsem, ssem, acc_sh):
        cid = lax.axis_index("c")
        sid = lax.axis_index("s")
        base = sid * rows

        @pl.when(cid == 1)
        def _body():
            pltpu.sync_copy(g_hbm.at[pl.ds(base, rows)], acc_sh.at[pl.ds(base, rows)])
            pltpu.sync_copy(src_hbm.at[sid], src_v)
            pltpu.sync_copy(dst_hbm.at[sid], dst_v)
            plsc.subcore_barrier()

            for b in range(2):
                pltpu.async_copy(g_hbm.at[src_v.at[b]], bufs[b], gsem[b])

            def edge_body(j0, _):
                for b in range(4):
                    j = j0 * 4 + b
                    b2 = (b + 2) % 4
                    pltpu.make_async_copy(g_hbm.at[src_v.at[j]], bufs[b], gsem[b]).wait()
                    pltpu.async_copy(bufs[b], acc_sh.at[dst_v.at[j]], ssem[b], add=True)

                    @pl.when(j >= 2)
                    def _():
                        pltpu.make_async_copy(
                            bufs[b2], acc_sh.at[dst_v.at[j - 2]], ssem[b2]
                        ).wait()

                    @pl.when(j + 2 < n_chunks)
                    def _():
                        pltpu.async_copy(g_hbm.at[src_v.at[j + 2]], bufs[b2], gsem[b2])
                return _

            lax.fori_loop(0, n_chunks // 4, edge_body, None)
            for b in (2, 3):
                j = n_chunks - 4 + b
                pltpu.make_async_copy(bufs[b], acc_sh.at[dst_v.at[j]], ssem[b]).wait()
            plsc.subcore_barrier()
            pltpu.sync_copy(acc_sh.at[pl.ds(base, rows)], acc_hbm.at[pl.ds(base, rows)])

    return agg_kernel


def _tc_linear(x, w, deg):
    def body(x_ref, w_ref, deg_ref, g_ref, d_ref):
        d_ref[...] = lax.rsqrt(deg_ref[...] + 1.0)
        h = jnp.dot(x_ref[...], w_ref[...], preferred_element_type=jnp.float32)
        g_ref[...] = h * d_ref[...]

    npad = x.shape[0]
    return pl.pallas_call(
        body,
        out_shape=(
            jax.ShapeDtypeStruct((npad, w.shape[1]), jnp.float32),
            jax.ShapeDtypeStruct((npad, 1), jnp.float32),
        ),
    )(x, w, deg)


def _tc_mid(acc, g1, dinv, b1, wcat):
    def body(a_ref, g_ref, d_ref, b_ref, w_ref, o_ref):
        h = jnp.maximum(a_ref[...] * d_ref[...] + b_ref[...], 0.0)
        o_ref[...] = (
            jnp.dot(h, w_ref[...], preferred_element_type=jnp.float32) * d_ref[...]
        )

    return pl.pallas_call(
        body,
        out_shape=jax.ShapeDtypeStruct(g1.shape, jnp.float32),
    )(acc, g1, dinv, b1, wcat)


def _tc_final(acc, g2, dinv, bcat):
    def body(a_ref, g_ref, d_ref, b_ref, o_ref):
        o_ref[...] = a_ref[...] * d_ref[...] + b_ref[...]

    return pl.pallas_call(
        body,
        out_shape=jax.ShapeDtypeStruct(g2.shape, jnp.float32),
    )(acc, g2, dinv, bcat)


def kernel(x, edge_index, W1, b1, W_mu, b_mu, W_logstd, b_logstd):
    n = x.shape[0]
    e = edge_index.shape[1]
    npad = ((n + 16 * 32 - 1) // (16 * 32)) * (16 * 32)

    ei = edge_index.astype(jnp.int32)
    src, dst = ei[0], ei[1]

    n_ch = -(-e // (NS * CHUNK))
    n_ch = ((n_ch + 3) // 4) * 4
    epad = NS * CHUNK * n_ch
    pad = jnp.full((epad - e,), n, jnp.int32)
    srcp = jnp.concatenate([src, pad]).reshape(NS, n_ch, CHUNK)
    dstp = jnp.concatenate([dst, pad]).reshape(NS, n_ch, CHUNK)

    n_chd = -(-e // (NS * CHUNK))
    epadd = NS * CHUNK * n_chd
    dstd = jnp.concatenate([dst, jnp.full((epadd - e,), n, jnp.int32)]).reshape(
        NS, n_chd, CHUNK
    )

    xp = jnp.pad(x, ((0, npad - n), (0, 0)))
    wcat = jnp.concatenate([W_mu, W_logstd], axis=1)
    bcat = jnp.concatenate([b_mu, b_logstd]).reshape(1, -1)

    deg = _make_deg_kernel(npad, n_chd)(dstd).reshape(npad, 1)
    agg = _make_agg_kernel(npad, W1.shape[1], n_ch)

    g1, dinv = _tc_linear(xp, W1, deg)
    acc1 = agg(g1, srcp, dstp)
    g2 = _tc_mid(acc1, g1, dinv, b1.reshape(1, -1), wcat)
    acc2 = agg(g2, srcp, dstp)
    outc = _tc_final(acc2, g2, dinv, bcat)

    o = W_mu.shape[1]
    return (outc[:n, :o], outc[:n, o:])


# speedup vs baseline: 2.8630x; 2.8597x over previous
import functools

import jax
import jax.numpy as jnp
from jax import lax
from jax.experimental import pallas as pl
from jax.experimental.pallas import tpu as pltpu
from jax.experimental.pallas import tpu_sc as plsc

NC = 2
NS = 16
LANES = 16
CHUNK = 128


def _make_deg_kernel(npad, n_chunks):
    rows = npad // NS
    mesh = plsc.VectorSubcoreMesh(core_axis_name="c", subcore_axis_name="s")

    @functools.partial(
        pl.kernel,
        mesh=mesh,
        out_type=jax.ShapeDtypeStruct((npad,), jnp.float32),
        scratch_types=[
            pltpu.VMEM((n_chunks, CHUNK), jnp.int32),
            pltpu.VMEM((CHUNK,), jnp.float32),
            pltpu.VMEM((rows,), jnp.float32),
            pltpu.VMEM_SHARED((npad,), jnp.float32),
        ],
    )
    def deg_kernel(dst_hbm, deg_hbm, idx_v, ones_v, buf_v, acc_sh):
        cid = lax.axis_index("c")
        sid = lax.axis_index("s")

        @pl.when(cid == 1)
        def _():
            def zero_body(k, _):
                buf_v[pl.ds(k * LANES, LANES)] = jnp.zeros((LANES,), jnp.float32)
                return _

            lax.fori_loop(0, rows // LANES, zero_body, None)
            pltpu.sync_copy(buf_v, acc_sh.at[pl.ds(sid * rows, rows)])

            def ones_body(k, _):
                ones_v[pl.ds(k * LANES, LANES)] = jnp.full((LANES,), 1.0, jnp.float32)
                return _

            lax.fori_loop(0, CHUNK // LANES, ones_body, None)
            pltpu.sync_copy(dst_hbm.at[sid], idx_v)
            plsc.subcore_barrier()

            def edge_body(j, _):
                pltpu.sync_copy(ones_v, acc_sh.at[idx_v.at[j]], add=True)
                return _

            lax.fori_loop(0, n_chunks, edge_body, None)
            plsc.subcore_barrier()
            pltpu.sync_copy(
                acc_sh.at[pl.ds(sid * rows, rows)],
                deg_hbm.at[pl.ds(sid * rows, rows)],
            )

    return deg_kernel


def _make_agg_kernel(npad, feat, n_chunks):
    rows = npad // NS
    assert n_chunks % 3 == 0 and n_chunks >= 6
    mesh = plsc.VectorSubcoreMesh(core_axis_name="c", subcore_axis_name="s")

    @functools.partial(
        pl.kernel,
        mesh=mesh,
        out_type=jax.ShapeDtypeStruct((NC, npad, feat), jnp.float32),
        scratch_types=[
            pltpu.VMEM((n_chunks, CHUNK), jnp.int32),
            pltpu.VMEM((n_chunks, CHUNK), jnp.int32),
            [pltpu.VMEM((CHUNK, feat), jnp.float32)] * 3,
            [pltpu.SemaphoreType.DMA] * 3,
            [pltpu.SemaphoreType.DMA] * 3,
            pltpu.VMEM_SHARED((npad, feat), jnp.float32),
            pltpu.VMEM_SHARED((npad, feat), jnp.float32),
        ],
        compiler_params=pltpu.CompilerParams(use_tc_tiling_on_sc=False),
    )
    def agg_kernel(
        g_hbm, src_hbm, dst_hbm, acc_hbm, src_v, dst_v, bufs, gsem, ssem, g_sh, acc_sh
    ):
        cid = lax.axis_index("c")
        sid = lax.axis_index("s")
        wid = cid * NS + sid
        base = sid * rows
        pltpu.sync_copy(g_hbm.at[pl.ds(base, rows)], g_sh.at[pl.ds(base, rows)])
        pltpu.sync_copy(g_hbm.at[pl.ds(base, rows)], acc_sh.at[pl.ds(base, rows)])
        pltpu.sync_copy(src_hbm.at[wid], src_v)
        pltpu.sync_copy(dst_hbm.at[wid], dst_v)
        plsc.subcore_barrier()

        for b in range(2):
            pltpu.async_copy(g_sh.at[src_v.at[b]], bufs[b], gsem[b])

        def edge_body(j0, _):
            for b in range(3):
                j = j0 * 3 + b
                b2 = (b + 2) % 3
                pltpu.make_async_copy(g_sh.at[src_v.at[j]], bufs[b], gsem[b]).wait()
                pltpu.async_copy(bufs[b], acc_sh.at[dst_v.at[j]], ssem[b], add=True)

                @pl.when(j >= 1)
                def _():
                    pltpu.make_async_copy(
                        bufs[b2], acc_sh.at[dst_v.at[j - 1]], ssem[b2]
                    ).wait()

                @pl.when(j + 2 < n_chunks)
                def _():
                    pltpu.async_copy(g_sh.at[src_v.at[j + 2]], bufs[b2], gsem[b2])
            return _

        lax.fori_loop(0, n_chunks // 3, edge_body, None)
        j = n_chunks - 1
        pltpu.make_async_copy(bufs[j % 3], acc_sh.at[dst_v.at[j]], ssem[j % 3]).wait()
        plsc.subcore_barrier()
        pltpu.sync_copy(acc_sh.at[pl.ds(base, rows)], acc_hbm.at[cid, pl.ds(base, rows)])

    return agg_kernel


def _tc_linear(x, w, deg):
    def body(x_ref, w_ref, deg_ref, g_ref, d_ref):
        d_ref[...] = lax.rsqrt(deg_ref[...] + 1.0)
        h = jnp.dot(x_ref[...], w_ref[...], preferred_element_type=jnp.float32)
        g_ref[...] = h * d_ref[...]

    npad = x.shape[0]
    return pl.pallas_call(
        body,
        out_shape=(
            jax.ShapeDtypeStruct((npad, w.shape[1]), jnp.float32),
            jax.ShapeDtypeStruct((npad, 1), jnp.float32),
        ),
    )(x, w, deg)


def _tc_mid(acc, g1, dinv, b1, wcat):
    def body(a_ref, g_ref, d_ref, b_ref, w_ref, o_ref):
        s = a_ref[0] + a_ref[1] - g_ref[...]
        h = jnp.maximum(s * d_ref[...] + b_ref[...], 0.0)
        o_ref[...] = (
            jnp.dot(h, w_ref[...], preferred_element_type=jnp.float32) * d_ref[...]
        )

    return pl.pallas_call(
        body,
        out_shape=jax.ShapeDtypeStruct(g1.shape, jnp.float32),
    )(acc, g1, dinv, b1, wcat)


def _tc_final(acc, g2, dinv, bcat):
    def body(a_ref, g_ref, d_ref, b_ref, o_ref):
        s = a_ref[0] + a_ref[1] - g_ref[...]
        o_ref[...] = s * d_ref[...] + b_ref[...]

    return pl.pallas_call(
        body,
        out_shape=jax.ShapeDtypeStruct(g2.shape, jnp.float32),
    )(acc, g2, dinv, bcat)


def kernel(x, edge_index, W1, b1, W_mu, b_mu, W_logstd, b_logstd):
    n = x.shape[0]
    e = edge_index.shape[1]
    npad = ((n + 16 * 32 - 1) // (16 * 32)) * (16 * 32)

    ei = edge_index.astype(jnp.int32)
    src, dst = ei[0], ei[1]

    n_ch = -(-e // (NC * NS * CHUNK))
    n_ch = ((n_ch + 2) // 3) * 3
    epad = NC * NS * CHUNK * n_ch
    pad = jnp.full((epad - e,), n, jnp.int32)
    srcp = jnp.concatenate([src, pad]).reshape(NC * NS, n_ch, CHUNK)
    dstp = jnp.concatenate([dst, pad]).reshape(NC * NS, n_ch, CHUNK)

    n_chd = -(-e // (NS * CHUNK))
    epadd = NS * CHUNK * n_chd
    dstd = jnp.concatenate([dst, jnp.full((epadd - e,), n, jnp.int32)]).reshape(
        NS, n_chd, CHUNK
    )

    xp = jnp.pad(x, ((0, npad - n), (0, 0)))
    wcat = jnp.concatenate([W_mu, W_logstd], axis=1)
    bcat = jnp.concatenate([b_mu, b_logstd]).reshape(1, -1)

    deg = _make_deg_kernel(npad, n_chd)(dstd).reshape(npad, 1)
    agg = _make_agg_kernel(npad, W1.shape[1], n_ch)

    g1, dinv = _tc_linear(xp, W1, deg)
    acc1 = agg(g1, srcp, dstp)
    g2 = _tc_mid(acc1, g1, dinv, b1.reshape(1, -1), wcat)
    acc2 = agg(g2, srcp, dstp)
    outc = _tc_final(acc2, g2, dinv, bcat)

    o = W_mu.shape[1]
    return (outc[:n, :o], outc[:n, o:])


# speedup vs baseline: 2.8817x; 1.0065x over previous
import functools

import jax
import jax.numpy as jnp
from jax import lax
from jax.experimental import pallas as pl
from jax.experimental.pallas import tpu as pltpu
from jax.experimental.pallas import tpu_sc as plsc

NC = 2
NS = 16
LANES = 16
CHUNK = 128


def _make_deg_kernel(npad, n_chunks):
    rows = npad // NS
    mesh = plsc.VectorSubcoreMesh(core_axis_name="c", subcore_axis_name="s")

    @functools.partial(
        pl.kernel,
        mesh=mesh,
        out_type=jax.ShapeDtypeStruct((NC, npad), jnp.float32),
        scratch_types=[
            pltpu.VMEM((n_chunks, CHUNK), jnp.int32),
            pltpu.VMEM((CHUNK,), jnp.float32),
            pltpu.VMEM((rows,), jnp.float32),
            pltpu.VMEM_SHARED((npad,), jnp.float32),
            pltpu.SemaphoreType.DMA,
        ],
    )
    def deg_kernel(dst_hbm, deg_hbm, idx_v, ones_v, buf_v, acc_sh, sem):
        cid = lax.axis_index("c")
        sid = lax.axis_index("s")
        wid = cid * NS + sid

        def zero_body(k, _):
            buf_v[pl.ds(k * LANES, LANES)] = jnp.zeros((LANES,), jnp.float32)
            return _

        lax.fori_loop(0, rows // LANES, zero_body, None)
        pltpu.sync_copy(buf_v, acc_sh.at[pl.ds(sid * rows, rows)])

        def ones_body(k, _):
            ones_v[pl.ds(k * LANES, LANES)] = jnp.full((LANES,), 1.0, jnp.float32)
            return _

        lax.fori_loop(0, CHUNK // LANES, ones_body, None)
        pltpu.sync_copy(dst_hbm.at[wid], idx_v)
        plsc.subcore_barrier()

        def edge_body(j, _):
            pltpu.async_copy(ones_v, acc_sh.at[idx_v.at[j]], sem, add=True)
            return _

        lax.fori_loop(0, n_chunks, edge_body, None)

        def drain_body(j, _):
            pltpu.make_async_copy(ones_v, acc_sh.at[idx_v.at[0]], sem).wait()
            return _

        lax.fori_loop(0, n_chunks, drain_body, None)
        plsc.subcore_barrier()
        pltpu.sync_copy(
            acc_sh.at[pl.ds(sid * rows, rows)],
            deg_hbm.at[cid, pl.ds(sid * rows, rows)],
        )

    return deg_kernel


def _make_agg_kernel(npad, feat, n_chunks):
    rows = npad // NS
    assert n_chunks % 3 == 0 and n_chunks >= 6
    mesh = plsc.VectorSubcoreMesh(core_axis_name="c", subcore_axis_name="s")

    @functools.partial(
        pl.kernel,
        mesh=mesh,
        out_type=jax.ShapeDtypeStruct((NC, npad, feat), jnp.float32),
        scratch_types=[
            pltpu.VMEM((n_chunks, CHUNK), jnp.int32),
            pltpu.VMEM((n_chunks, CHUNK), jnp.int32),
            [pltpu.VMEM((CHUNK, feat), jnp.float32)] * 3,
            [pltpu.SemaphoreType.DMA] * 3,
            [pltpu.SemaphoreType.DMA] * 3,
            pltpu.VMEM_SHARED((npad, feat), jnp.float32),
            pltpu.VMEM_SHARED((npad, feat), jnp.float32),
        ],
        compiler_params=pltpu.CompilerParams(use_tc_tiling_on_sc=False),
    )
    def agg_kernel(
        g_hbm, src_hbm, dst_hbm, acc_hbm, src_v, dst_v, bufs, gsem, ssem, g_sh, acc_sh
    ):
        cid = lax.axis_index("c")
        sid = lax.axis_index("s")
        wid = cid * NS + sid
        base = sid * rows
        pltpu.sync_copy(g_hbm.at[pl.ds(base, rows)], g_sh.at[pl.ds(base, rows)])
        pltpu.sync_copy(g_hbm.at[pl.ds(base, rows)], acc_sh.at[pl.ds(base, rows)])
        pltpu.sync_copy(src_hbm.at[wid], src_v)
        pltpu.sync_copy(dst_hbm.at[wid], dst_v)
        plsc.subcore_barrier()

        for b in range(2):
            pltpu.async_copy(g_sh.at[src_v.at[b]], bufs[b], gsem[b])

        def edge_body(j0, _):
            for b in range(3):
                j = j0 * 3 + b
                b2 = (b + 2) % 3
                pltpu.make_async_copy(g_sh.at[src_v.at[j]], bufs[b], gsem[b]).wait()
                pltpu.async_copy(bufs[b], acc_sh.at[dst_v.at[j]], ssem[b], add=True)

                @pl.when(j >= 1)
                def _():
                    pltpu.make_async_copy(
                        bufs[b2], acc_sh.at[dst_v.at[j - 1]], ssem[b2]
                    ).wait()

                @pl.when(j + 2 < n_chunks)
                def _():
                    pltpu.async_copy(g_sh.at[src_v.at[j + 2]], bufs[b2], gsem[b2])
            return _

        lax.fori_loop(0, n_chunks // 3, edge_body, None)
        j = n_chunks - 1
        pltpu.make_async_copy(bufs[j % 3], acc_sh.at[dst_v.at[j]], ssem[j % 3]).wait()
        plsc.subcore_barrier()
        pltpu.sync_copy(acc_sh.at[pl.ds(base, rows)], acc_hbm.at[cid, pl.ds(base, rows)])

    return agg_kernel


def _tc_linear(x, w, deg):
    def body(x_ref, w_ref, deg_ref, g_ref, d_ref):
        d_ref[...] = lax.rsqrt(deg_ref[0] + deg_ref[1] + 1.0)
        h = jnp.dot(x_ref[...], w_ref[...], preferred_element_type=jnp.float32)
        g_ref[...] = h * d_ref[...]

    npad = x.shape[0]
    return pl.pallas_call(
        body,
        out_shape=(
            jax.ShapeDtypeStruct((npad, w.shape[1]), jnp.float32),
            jax.ShapeDtypeStruct((npad, 1), jnp.float32),
        ),
    )(x, w, deg)


def _tc_mid(acc, g1, dinv, b1, wcat):
    def body(a_ref, g_ref, d_ref, b_ref, w_ref, o_ref):
        s = a_ref[0] + a_ref[1] - g_ref[...]
        h = jnp.maximum(s * d_ref[...] + b_ref[...], 0.0)
        o_ref[...] = (
            jnp.dot(h, w_ref[...], preferred_element_type=jnp.float32) * d_ref[...]
        )

    return pl.pallas_call(
        body,
        out_shape=jax.ShapeDtypeStruct(g1.shape, jnp.float32),
    )(acc, g1, dinv, b1, wcat)


def _tc_final(acc, g2, dinv, bcat, n, o):
    def body(a_ref, g_ref, d_ref, b_ref, mu_ref, ls_ref):
        s = a_ref[0] + a_ref[1] - g_ref[...]
        full = s * d_ref[...] + b_ref[...]
        mu_ref[...] = full[:n, :o]
        ls_ref[...] = full[:n, o:]

    return pl.pallas_call(
        body,
        out_shape=(
            jax.ShapeDtypeStruct((n, o), jnp.float32),
            jax.ShapeDtypeStruct((n, o), jnp.float32),
        ),
    )(acc, g2, dinv, bcat)


def kernel(x, edge_index, W1, b1, W_mu, b_mu, W_logstd, b_logstd):
    n = x.shape[0]
    e = edge_index.shape[1]
    npad = ((n + 16 * 32 - 1) // (16 * 32)) * (16 * 32)

    ei = edge_index.astype(jnp.int32)
    src, dst = ei[0], ei[1]

    n_ch = -(-e // (NC * NS * CHUNK))
    n_ch = ((n_ch + 2) // 3) * 3
    epad = NC * NS * CHUNK * n_ch
    pad = jnp.full((epad - e,), n, jnp.int32)
    srcp = jnp.concatenate([src, pad]).reshape(NC * NS, n_ch, CHUNK)
    dstp = jnp.concatenate([dst, pad]).reshape(NC * NS, n_ch, CHUNK)

    xp = jnp.pad(x, ((0, npad - n), (0, 0)))
    wcat = jnp.concatenate([W_mu, W_logstd], axis=1)
    bcat = jnp.concatenate([b_mu, b_logstd]).reshape(1, -1)

    deg = _make_deg_kernel(npad, n_ch)(dstp).reshape(NC, npad, 1)
    agg = _make_agg_kernel(npad, W1.shape[1], n_ch)

    g1, dinv = _tc_linear(xp, W1, deg)
    acc1 = agg(g1, srcp, dstp)
    g2 = _tc_mid(acc1, g1, dinv, b1.reshape(1, -1), wcat)
    acc2 = agg(g2, srcp, dstp)
    return _tc_final(acc2, g2, dinv, bcat, n, W_mu.shape[1])


# speedup vs baseline: 3.1566x; 1.0954x over previous
import functools

import jax
import jax.numpy as jnp
from jax import lax
from jax.experimental import pallas as pl
from jax.experimental.pallas import tpu as pltpu
from jax.experimental.pallas import tpu_sc as plsc

NC = 2
NS = 16
LANES = 16
CHUNK = 128


def _make_deg_kernel(npad, n_chunks):
    rows = npad // NS
    cpw = -(-n_chunks // (NC * NS))
    mesh = plsc.VectorSubcoreMesh(core_axis_name="c", subcore_axis_name="s")

    @functools.partial(
        pl.kernel,
        mesh=mesh,
        out_type=jax.ShapeDtypeStruct((NC, npad), jnp.float32),
        scratch_types=[
            pltpu.VMEM((cpw, CHUNK), jnp.int32),
            pltpu.VMEM((CHUNK,), jnp.float32),
            pltpu.VMEM((rows,), jnp.float32),
            pltpu.VMEM_SHARED((npad,), jnp.float32),
            pltpu.SemaphoreType.DMA,
        ],
        compiler_params=pltpu.CompilerParams(use_tc_tiling_on_sc=False),
    )
    def deg_kernel(dst_hbm, deg_hbm, idx_v, ones_v, buf_v, acc_sh, sem):
        cid = lax.axis_index("c")
        sid = lax.axis_index("s")
        wid = cid * NS + sid
        lo = n_chunks * wid // (NC * NS)
        cnt = n_chunks * (wid + 1) // (NC * NS) - lo

        def zero_body(k, _):
            buf_v[pl.ds(k * LANES, LANES)] = jnp.zeros((LANES,), jnp.float32)
            return _

        lax.fori_loop(0, rows // LANES, zero_body, None)
        pltpu.sync_copy(buf_v, acc_sh.at[pl.ds(sid * rows, rows)])

        def ones_body(k, _):
            ones_v[pl.ds(k * LANES, LANES)] = jnp.full((LANES,), 1.0, jnp.float32)
            return _

        lax.fori_loop(0, CHUNK // LANES, ones_body, None)
        pltpu.sync_copy(dst_hbm.at[pl.ds(lo, cpw)], idx_v)
        plsc.subcore_barrier()

        def edge_body(j, _):
            pltpu.async_copy(ones_v, acc_sh.at[idx_v.at[j]], sem, add=True)
            return _

        lax.fori_loop(0, cnt, edge_body, None)

        def drain_body(j, _):
            pltpu.make_async_copy(ones_v, acc_sh.at[idx_v.at[0]], sem).wait()
            return _

        lax.fori_loop(0, cnt, drain_body, None)
        plsc.subcore_barrier()
        pltpu.sync_copy(
            acc_sh.at[pl.ds(sid * rows, rows)],
            deg_hbm.at[cid, pl.ds(sid * rows, rows)],
        )

    return deg_kernel


def _make_agg_kernel(npad, feat, n_chunks):
    rows = npad // NS
    half = feat // NC
    cpw = -(-n_chunks // NS)
    steps = 4 * (-(-(cpw + 2) // 4))
    mesh = plsc.VectorSubcoreMesh(core_axis_name="c", subcore_axis_name="s")

    @functools.partial(
        pl.kernel,
        mesh=mesh,
        out_type=jax.ShapeDtypeStruct((npad, feat), jnp.float32),
        scratch_types=[
            pltpu.VMEM((cpw, CHUNK), jnp.int32),
            pltpu.VMEM((cpw, CHUNK), jnp.int32),
            [pltpu.VMEM((CHUNK, half), jnp.float32)] * 4,
            [pltpu.SemaphoreType.DMA] * 4,
            [pltpu.SemaphoreType.DMA] * 4,
            pltpu.VMEM_SHARED((npad, half), jnp.float32),
            pltpu.VMEM_SHARED((npad, half), jnp.float32),
        ],
        compiler_params=pltpu.CompilerParams(use_tc_tiling_on_sc=False),
    )
    def agg_kernel(
        g_hbm, src_hbm, dst_hbm, acc_hbm, src_v, dst_v, bufs, gsem, ssem, g_sh, acc_sh
    ):
        cid = lax.axis_index("c")
        sid = lax.axis_index("s")
        lo = n_chunks * sid // NS
        cnt = n_chunks * (sid + 1) // NS - lo
        base = sid * rows
        cols = pl.ds(cid * half, half)
        pltpu.sync_copy(g_hbm.at[pl.ds(base, rows), cols], g_sh.at[pl.ds(base, rows)])
        pltpu.sync_copy(g_hbm.at[pl.ds(base, rows), cols], acc_sh.at[pl.ds(base, rows)])
        pltpu.sync_copy(src_hbm.at[pl.ds(lo, cpw)], src_v)
        pltpu.sync_copy(dst_hbm.at[pl.ds(lo, cpw)], dst_v)
        plsc.subcore_barrier()

        for b in range(2):
            pltpu.async_copy(g_sh.at[src_v.at[b]], bufs[b], gsem[b])

        def edge_body(j0, _):
            for b in range(4):
                j = j0 * 4 + b
                b2 = (b + 2) % 4

                @pl.when(j < cnt)
                def _():
                    pltpu.make_async_copy(g_sh.at[src_v.at[j]], bufs[b], gsem[b]).wait()
                    pltpu.async_copy(bufs[b], acc_sh.at[dst_v.at[j]], ssem[b], add=True)

                @pl.when((j >= 2) & (j < cnt + 2))
                def _():
                    pltpu.make_async_copy(
                        bufs[b2], acc_sh.at[dst_v.at[j - 2]], ssem[b2]
                    ).wait()

                @pl.when(j + 2 < cnt)
                def _():
                    pltpu.async_copy(g_sh.at[src_v.at[j + 2]], bufs[b2], gsem[b2])
            return _

        lax.fori_loop(0, steps // 4, edge_body, None)
        plsc.subcore_barrier()
        pltpu.sync_copy(
            acc_sh.at[pl.ds(base, rows)], acc_hbm.at[pl.ds(base, rows), cols]
        )

    return agg_kernel


def _tc_linear(x, w, deg):
    def body(x_ref, w_ref, deg_ref, g_ref, d_ref):
        d_ref[...] = lax.rsqrt(deg_ref[0] + deg_ref[1] + 1.0)
        h = jnp.dot(x_ref[...], w_ref[...], preferred_element_type=jnp.float32)
        g_ref[...] = h * d_ref[...]

    npad = x.shape[0]
    return pl.pallas_call(
        body,
        out_shape=(
            jax.ShapeDtypeStruct((npad, w.shape[1]), jnp.float32),
            jax.ShapeDtypeStruct((npad, 1), jnp.float32),
        ),
    )(x, w, deg)


def _tc_mid(acc, dinv, b1, wcat):
    def body(a_ref, d_ref, b_ref, w_ref, o_ref):
        h = jnp.maximum(a_ref[...] * d_ref[...] + b_ref[...], 0.0)
        o_ref[...] = (
            jnp.dot(h, w_ref[...], preferred_element_type=jnp.float32) * d_ref[...]
        )

    return pl.pallas_call(
        body,
        out_shape=jax.ShapeDtypeStruct(acc.shape, jnp.float32),
    )(acc, dinv, b1, wcat)


def _tc_final(acc, dinv, bcat, n, o):
    def body(a_ref, d_ref, b_ref, mu_ref, ls_ref):
        full = a_ref[...] * d_ref[...] + b_ref[...]
        mu_ref[...] = full[:n, :o]
        ls_ref[...] = full[:n, o:]

    return pl.pallas_call(
        body,
        out_shape=(
            jax.ShapeDtypeStruct((n, o), jnp.float32),
            jax.ShapeDtypeStruct((n, o), jnp.float32),
        ),
    )(acc, dinv, bcat)


def kernel(x, edge_index, W1, b1, W_mu, b_mu, W_logstd, b_logstd):
    n = x.shape[0]
    e = edge_index.shape[1]
    npad = ((n + 16 * 32 - 1) // (16 * 32)) * (16 * 32)

    ei = edge_index.astype(jnp.int32)
    src, dst = ei[0], ei[1]
    if e % CHUNK:
        pad = jnp.full((CHUNK - e % CHUNK,), n, jnp.int32)
        src = jnp.concatenate([src, pad])
        dst = jnp.concatenate([dst, pad])
    n_ch = src.shape[0] // CHUNK
    srcc = src.reshape(n_ch, CHUNK)
    dstc = dst.reshape(n_ch, CHUNK)

    xp = jnp.pad(x, ((0, npad - n), (0, 0)))
    wcat = jnp.concatenate([W_mu, W_logstd], axis=1)
    bcat = jnp.concatenate([b_mu, b_logstd]).reshape(1, -1)

    deg = _make_deg_kernel(npad, n_ch)(dstc).reshape(NC, npad, 1)
    agg = _make_agg_kernel(npad, W1.shape[1], n_ch)

    g1, dinv = _tc_linear(xp, W1, deg)
    acc1 = agg(g1, srcc, dstc)
    g2 = _tc_mid(acc1, dinv, b1.reshape(1, -1), wcat)
    acc2 = agg(g2, srcc, dstc)
    return _tc_final(acc2, dinv, bcat, n, W_mu.shape[1])
